# Initial kernel scaffold; baseline (speedup 1.0000x reference)
import jax
import jax.numpy as jnp
from jax.experimental import pallas as pl


def kernel(x, table, W1, b1, W2, b2):
    raise NotImplementedError("write your pallas kernel here")


# speedup vs baseline: 36.6433x; 36.6433x over previous
import functools

import jax
import jax.numpy as jnp
from jax import lax
from jax.experimental import pallas as pl
from jax.experimental.pallas import tpu as pltpu
from jax.experimental.pallas import tpu_sc as plsc

VOCAB = 100000
EMB_DIM = 128
UNITS = 512


_VBLK = 4000


def _lut_body(tab_ref, w1_ref, b1_ref, w2_ref, b2_ref, out_ref):
    h = jnp.maximum(tab_ref[...], 0.0)
    a = jnp.dot(h, w1_ref[...], preferred_element_type=jnp.float32) + b1_ref[...]
    a = jnp.maximum(a, 0.0)
    out_ref[...] = jnp.dot(a, w2_ref[...], preferred_element_type=jnp.float32) + b2_ref[...]


def _build_lut(table, W1, b1, W2, b2):
    b1r = b1.reshape(1, UNITS)
    b2r = b2.reshape(1, 1)
    grid = VOCAB // _VBLK
    out = pl.pallas_call(
        _lut_body,
        grid=(grid,),
        in_specs=[
            pl.BlockSpec((_VBLK, EMB_DIM), lambda i: (i, 0)),
            pl.BlockSpec((EMB_DIM, UNITS), lambda i: (0, 0)),
            pl.BlockSpec((1, UNITS), lambda i: (0, 0)),
            pl.BlockSpec((UNITS, 1), lambda i: (0, 0)),
            pl.BlockSpec((1, 1), lambda i: (0, 0)),
        ],
        out_specs=pl.BlockSpec((_VBLK, 1), lambda i: (i, 0)),
        out_shape=jax.ShapeDtypeStruct((VOCAB, 1), jnp.float32),
    )(table, W1, b1r, W2, b2r)
    return out.reshape(VOCAB)


_NC = 2
_NS = 16
_NW = _NC * _NS


def _gather_body(n_per_w, lut_hbm, idx_hbm, out_hbm, idx_v, out_v, sem):
    wid = lax.axis_index("s") * _NC + lax.axis_index("c")
    base = wid * n_per_w
    pltpu.sync_copy(idx_hbm.at[pl.ds(base, n_per_w)], idx_v)
    pltpu.async_copy(lut_hbm.at[idx_v], out_v, sem).wait()
    pltpu.sync_copy(out_v, out_hbm.at[pl.ds(base, n_per_w)])


def _gather(lut, xf):
    n = xf.shape[0]
    n_per_w = n // _NW
    mesh = plsc.VectorSubcoreMesh(core_axis_name="c", subcore_axis_name="s")
    return pl.kernel(
        functools.partial(_gather_body, n_per_w),
        out_type=jax.ShapeDtypeStruct((n,), jnp.float32),
        mesh=mesh,
        scratch_types=[
            pltpu.VMEM((n_per_w,), jnp.int32),
            pltpu.VMEM((n_per_w,), jnp.float32),
            pltpu.SemaphoreType.DMA,
        ],
    )(lut, xf)


def kernel(x, table, W1, b1, W2, b2):
    lut = _build_lut(table, W1, b1, W2, b2)
    B, F = x.shape
    out_flat = _gather(lut, x.reshape(B * F))
    return out_flat.reshape(B, F, 1)


# speedup vs baseline: 38.1362x; 1.0407x over previous
import functools

import jax
import jax.numpy as jnp
from jax import lax
from jax.experimental import pallas as pl
from jax.experimental.pallas import tpu as pltpu
from jax.experimental.pallas import tpu_sc as plsc

VOCAB = 100000
EMB_DIM = 128
UNITS = 512


_VPAD = 100352
_VBLK = 7168
_GRID = _VPAD // _VBLK


def _lut_body(tab_ref, w1_ref, b1_ref, w2_ref, b2_ref, out_ref):
    h = jnp.maximum(tab_ref[...], 0.0).astype(jnp.bfloat16)
    a = jnp.dot(h, w1_ref[...], preferred_element_type=jnp.float32) + b1_ref[...]
    a = jnp.maximum(a, 0.0)
    o = jnp.dot(a, w2_ref[...], preferred_element_type=jnp.float32) + b2_ref[...]
    out_ref[...] = o.reshape(_VBLK // 128, 128)


def _build_lut(table, W1, b1, W2, b2):
    b1r = b1.reshape(1, UNITS)
    b2r = b2.reshape(1, 1)
    out = pl.pallas_call(
        _lut_body,
        grid=(_GRID,),
        in_specs=[
            pl.BlockSpec((_VBLK, EMB_DIM), lambda i: (i, 0)),
            pl.BlockSpec((EMB_DIM, UNITS), lambda i: (0, 0)),
            pl.BlockSpec((1, UNITS), lambda i: (0, 0)),
            pl.BlockSpec((UNITS, 1), lambda i: (0, 0)),
            pl.BlockSpec((1, 1), lambda i: (0, 0)),
        ],
        out_specs=pl.BlockSpec((_VBLK // 128, 128), lambda i: (i, 0)),
        out_shape=jax.ShapeDtypeStruct((_VPAD // 128, 128), jnp.float32),
    )(table, W1.astype(jnp.bfloat16), b1r, W2, b2r)
    return out.reshape(_VPAD)


_NC = 2
_NS = 16
_NW = _NC * _NS


def _gather_body(n_per_w, lut_hbm, idx_hbm, out_hbm, idx_v, out_v, sem):
    wid = lax.axis_index("s") * _NC + lax.axis_index("c")
    base = wid * n_per_w
    pltpu.sync_copy(idx_hbm.at[pl.ds(base, n_per_w)], idx_v)
    pltpu.async_copy(lut_hbm.at[idx_v], out_v, sem).wait()
    pltpu.sync_copy(out_v, out_hbm.at[pl.ds(base, n_per_w)])


def _gather(lut, xf):
    n = xf.shape[0]
    n_per_w = n // _NW
    mesh = plsc.VectorSubcoreMesh(core_axis_name="c", subcore_axis_name="s")
    return pl.kernel(
        functools.partial(_gather_body, n_per_w),
        out_type=jax.ShapeDtypeStruct((n,), jnp.float32),
        mesh=mesh,
        scratch_types=[
            pltpu.VMEM((n_per_w,), jnp.int32),
            pltpu.VMEM((n_per_w,), jnp.float32),
            pltpu.SemaphoreType.DMA,
        ],
    )(lut, xf)


def kernel(x, table, W1, b1, W2, b2):
    lut = _build_lut(table, W1, b1, W2, b2)
    B, F = x.shape
    out_flat = _gather(lut, x.reshape(B * F))
    return out_flat.reshape(B, F, 1)


# speedup vs baseline: 55.1249x; 1.4455x over previous
import functools

import jax
import jax.numpy as jnp
from jax import lax
from jax.experimental import pallas as pl
from jax.experimental.pallas import tpu as pltpu
from jax.experimental.pallas import tpu_sc as plsc

VOCAB = 100000
EMB_DIM = 128
UNITS = 512


_VPAD = 100352
_VBLK = 7168
_GRID = _VPAD // _VBLK


def _lut_body(tab_ref, w1_ref, b1_ref, w2_ref, b2_ref, out_ref):
    h = jnp.maximum(tab_ref[...], 0.0).astype(jnp.bfloat16)
    a = jnp.dot(h, w1_ref[...], preferred_element_type=jnp.float32) + b1_ref[...]
    a = jnp.maximum(a, 0.0)
    o = jnp.dot(a, w2_ref[...], preferred_element_type=jnp.float32) + b2_ref[...]
    out_ref[...] = o.reshape(_VBLK // 128, 128)


def _build_lut(table, W1, b1, W2, b2):
    b1r = b1.reshape(1, UNITS)
    b2r = b2.reshape(1, 1)
    out = pl.pallas_call(
        _lut_body,
        grid=(_GRID,),
        in_specs=[
            pl.BlockSpec((_VBLK, EMB_DIM), lambda i: (i, 0)),
            pl.BlockSpec((EMB_DIM, UNITS), lambda i: (0, 0)),
            pl.BlockSpec((1, UNITS), lambda i: (0, 0)),
            pl.BlockSpec((UNITS, 1), lambda i: (0, 0)),
            pl.BlockSpec((1, 1), lambda i: (0, 0)),
        ],
        out_specs=pl.BlockSpec((_VBLK // 128, 128), lambda i: (i, 0)),
        out_shape=jax.ShapeDtypeStruct((_VPAD // 128, 128), jnp.float32),
    )(table, W1.astype(jnp.bfloat16), b1r, W2, b2r)
    return out.reshape(_VPAD)


_NC = 2
_NS = 16
_NW = _NC * _NS


def _gather_body(n_per_w, lut_hbm, idx_hbm, out_hbm, lut_sp, idx_v, out_v, sem):
    sid = lax.axis_index("s")
    wid = sid * _NC + lax.axis_index("c")
    seg = _VPAD // _NS
    pltpu.sync_copy(lut_hbm.at[pl.ds(sid * seg, seg)],
                    lut_sp.at[pl.ds(sid * seg, seg)])
    plsc.subcore_barrier()
    base = wid * n_per_w
    pltpu.sync_copy(idx_hbm.at[pl.ds(base, n_per_w)], idx_v)
    pltpu.async_copy(lut_sp.at[idx_v], out_v, sem).wait()
    pltpu.sync_copy(out_v, out_hbm.at[pl.ds(base, n_per_w)])


def _gather(lut, xf):
    n = xf.shape[0]
    n_per_w = n // _NW
    mesh = plsc.VectorSubcoreMesh(core_axis_name="c", subcore_axis_name="s")
    return pl.kernel(
        functools.partial(_gather_body, n_per_w),
        out_type=jax.ShapeDtypeStruct((n,), jnp.float32),
        mesh=mesh,
        scratch_types=[
            pltpu.VMEM_SHARED((_VPAD,), jnp.float32),
            pltpu.VMEM((n_per_w,), jnp.int32),
            pltpu.VMEM((n_per_w,), jnp.float32),
            pltpu.SemaphoreType.DMA,
        ],
    )(lut, xf)


def kernel(x, table, W1, b1, W2, b2):
    lut = _build_lut(table, W1, b1, W2, b2)
    B, F = x.shape
    out_flat = _gather(lut, x.reshape(B * F))
    return out_flat.reshape(B, F, 1)


# speedup vs baseline: 59.6756x; 1.0826x over previous
import functools

import jax
import jax.numpy as jnp
from jax import lax
from jax.experimental import pallas as pl
from jax.experimental.pallas import tpu as pltpu
from jax.experimental.pallas import tpu_sc as plsc

VOCAB = 100000
EMB_DIM = 128
UNITS = 512


_VPAD = 100352
_VBLK = 7168
_GRID = _VPAD // _VBLK


def _lut_body(tab_ref, w1_ref, b1_ref, w2_ref, b2_ref, out_ref):
    h = jnp.maximum(tab_ref[...], 0.0).astype(jnp.bfloat16)
    a = jnp.dot(h, w1_ref[...], preferred_element_type=jnp.float32) + b1_ref[...]
    a = jnp.maximum(a, 0.0)
    o = jnp.dot(a, w2_ref[...], preferred_element_type=jnp.float32) + b2_ref[...]
    out_ref[...] = o.reshape(_VBLK // 128, 128)


def _build_lut(table, W1, b1, W2, b2):
    b1r = b1.reshape(1, UNITS)
    b2r = b2.reshape(1, 1)
    out = pl.pallas_call(
        _lut_body,
        grid=(_GRID,),
        in_specs=[
            pl.BlockSpec((_VBLK, EMB_DIM), lambda i: (i, 0)),
            pl.BlockSpec((EMB_DIM, UNITS), lambda i: (0, 0)),
            pl.BlockSpec((1, UNITS), lambda i: (0, 0)),
            pl.BlockSpec((UNITS, 1), lambda i: (0, 0)),
            pl.BlockSpec((1, 1), lambda i: (0, 0)),
        ],
        out_specs=pl.BlockSpec((_VBLK // 128, 128), lambda i: (i, 0)),
        out_shape=jax.ShapeDtypeStruct((_VPAD // 128, 128), jnp.float32),
    )(table, W1.astype(jnp.bfloat16), b1r, W2, b2r)
    return out.reshape(_VPAD)


_NC = 2
_NS = 16
_NW = _NC * _NS


_CH = 128
_K = 8


def _gather_body(rows_per_w, n_fields, lut_hbm, idx_hbm, out_hbm,
                 lut_sp, idx_v, out_v, sem):
    sid = lax.axis_index("s")
    wid = sid * _NC + lax.axis_index("c")
    seg = _VPAD // _NS
    pltpu.sync_copy(lut_hbm.at[pl.ds(sid * seg, seg)],
                    lut_sp.at[pl.ds(sid * seg, seg)])
    plsc.subcore_barrier()
    rb = wid * rows_per_w

    def chunk_body(c, _):
        r0 = rb + c * _CH
        pltpu.sync_copy(idx_hbm.at[pl.ds(r0, _CH), :], idx_v)

        def grp_body(g, _):
            handles = [
                pltpu.async_copy(lut_sp.at[idx_v.at[g * _K + k]],
                                 out_v.at[g * _K + k], sem)
                for k in range(_K)
            ]
            for h in handles:
                h.wait()
            return 0

        lax.fori_loop(0, _CH // _K, grp_body, 0)
        pltpu.sync_copy(out_v, out_hbm.at[pl.ds(r0, _CH), :])
        return 0

    lax.fori_loop(0, rows_per_w // _CH, chunk_body, 0)


def _gather(lut, x):
    B, F = x.shape
    rows_per_w = B // _NW
    mesh = plsc.VectorSubcoreMesh(core_axis_name="c", subcore_axis_name="s")
    return pl.kernel(
        functools.partial(_gather_body, rows_per_w, F),
        out_type=jax.ShapeDtypeStruct((B, F), jnp.float32),
        mesh=mesh,
        scratch_types=[
            pltpu.VMEM_SHARED((_VPAD,), jnp.float32),
            pltpu.VMEM((_CH, F), jnp.int32),
            pltpu.VMEM((_CH, F), jnp.float32),
            pltpu.SemaphoreType.DMA,
        ],
    )(lut, x)


def kernel(x, table, W1, b1, W2, b2):
    lut = _build_lut(table, W1, b1, W2, b2)
    B, F = x.shape
    out2d = _gather(lut, x)
    return out2d.reshape(B, F, 1)
